# baseline (speedup 1.0000x reference)
import functools
import jax
import jax.numpy as jnp
from jax import lax
from jax.experimental import pallas as pl
from jax.experimental.pallas import tpu as pltpu, tpu_sc as plsc

B, D, T = 4, 1024, 2048
TW = 512
DW = 128
TP = 256
DC = 16
NCH = DW // DC


def _sc_body(q_hbm, pos_hbm, out_hbm, pos_v, q_v, o_v, sem_p, sem_q, sem_o):
    c = lax.axis_index("c")
    s = lax.axis_index("s")
    tix = s % 4
    dix = (s // 4) + c * 4
    t0 = tix * TW
    d0 = dix * DW

    def start_q(buf, i, th):
        return [
            pltpu.async_copy(
                q_hbm.at[b, pl.ds(d0 + i * DC, DC), pl.ds(th, TP)],
                q_v.at[buf, b],
                sem_q,
            )
            for b in range(B)
        ]

    def compute(buf, i):
        @plsc.parallel_loop(0, (TP // 16) * DC, unroll=4)
        def body(k):
            tg = k // DC
            d_local = k % DC
            idx_t = lax.iota(jnp.int32, 16) + tg * 16
            idx_d = jnp.full((16,), i * DC + d_local, jnp.int32)
            pos_reg = plsc.load_gather(pos_v, [idx_t, idx_d])
            for b in range(B):
                o_v[buf, b, d_local, pl.ds(tg * 16, 16)] = (
                    q_v[buf, b, d_local, pl.ds(tg * 16, 16)] + pos_reg
                )

    def start_o(buf, i, th):
        return [
            pltpu.async_copy(
                o_v.at[buf, b],
                out_hbm.at[b, pl.ds(d0 + i * DC, DC), pl.ds(th, TP)],
                sem_o,
            )
            for b in range(B)
        ]

    for h in range(TW // TP):
        th = t0 + h * TP
        ph = pltpu.async_copy(
            pos_hbm.at[pl.ds(th, TP), pl.ds(d0, DW)], pos_v, sem_p
        )
        q_pend = start_q(0, 0, th)
        ph.wait()
        o_pend = [None, None]
        for i in range(NCH):
            buf = i % 2
            nxt = q_pend
            if i + 1 < NCH:
                q_pend = start_q(1 - buf, i + 1, th)
            for hq in nxt:
                hq.wait()
            if o_pend[buf] is not None:
                for ho in o_pend[buf]:
                    ho.wait()
            compute(buf, i)
            o_pend[buf] = start_o(buf, i, th)
        for pend in o_pend:
            if pend is not None:
                for ho in pend:
                    ho.wait()


def kernel(q, pos_weight):
    mesh = plsc.VectorSubcoreMesh(core_axis_name="c", subcore_axis_name="s")
    k = functools.partial(
        pl.kernel,
        mesh=mesh,
        out_type=jax.ShapeDtypeStruct((B, D, T), jnp.float32),
        scratch_types=[
            pltpu.VMEM((TP, DW), jnp.float32),
            pltpu.VMEM((2, B, DC, TP), jnp.float32),
            pltpu.VMEM((2, B, DC, TP), jnp.float32),
            pltpu.SemaphoreType.DMA,
            pltpu.SemaphoreType.DMA,
            pltpu.SemaphoreType.DMA,
        ],
        compiler_params=pltpu.CompilerParams(needs_layout_passes=False),
    )(_sc_body)
    return k(q, pos_weight)
